# baseline (speedup 1.0000x reference)
import functools

import jax
import jax.numpy as jnp
from jax.experimental import pallas as pl
from jax.experimental.pallas import tpu as pltpu

EPS = 1e-5
F32 = jnp.float32
BF16 = jnp.bfloat16


def _ln_rows(xr, w, b):
    m = jnp.mean(xr, axis=-1, keepdims=True)
    xc = xr - m
    v = jnp.mean(xc * xc, axis=-1, keepdims=True)
    return xc * jax.lax.rsqrt(v + EPS) * w + b


def _ln_body(x_ref, lw_ref, lb_ref, o_ref):
    o_ref[...] = _ln_rows(x_ref[...], lw_ref[...], lb_ref[...]).astype(BF16)


def _qkv_body(xn_ref, wq_ref, wk_ref, wv_ref, q_ref, k_ref, v_ref):
    xn = xn_ref[...]
    q_ref[...] = jnp.dot(xn, wq_ref[...],
                         preferred_element_type=F32).astype(BF16)
    k_ref[...] = jnp.dot(xn, wk_ref[...],
                         preferred_element_type=F32).astype(BF16)
    v_ref[...] = jnp.dot(xn, wv_ref[...],
                         preferred_element_type=F32).astype(BF16)


def _attn_body(nc, dh, x_ref, q_ref, k_ref, v_ref, g_ref, s0_ref, gf_ref,
               y_ref, fs_ref, st_ref):
    c = pl.program_id(1)

    @pl.when(c == 0)
    def _():
        st_ref[...] = s0_ref[...]

    cs = q_ref.shape[0]
    row = jax.lax.broadcasted_iota(jnp.int32, (cs, cs), 0)
    col = jax.lax.broadcasted_iota(jnp.int32, (cs, cs), 1)
    causal = row >= col
    tmul = (c * cs + 1 + jax.lax.broadcasted_iota(jnp.int32, (cs, 1), 0)
            ).astype(F32)

    outs = []
    for j in range(2):
        qj = q_ref[:, j * dh:(j + 1) * dh]
        kj = k_ref[:, j * dh:(j + 1) * dh]
        vj = v_ref[:, j * dh:(j + 1) * dh]
        s = jax.lax.dot_general(qj, vj, (((1,), (1,)), ((), ())),
                                preferred_element_type=F32)
        sm = jnp.where(causal, s, 0.0).astype(BF16)
        intra = jnp.dot(sm, kj, preferred_element_type=F32)
        inter = jax.lax.dot_general(qj, st_ref[j].astype(BF16),
                                    (((1,), (1,)), ((), ())),
                                    preferred_element_type=F32)
        qs = jnp.sum(qj.astype(F32), axis=1, keepdims=True)
        og = (tmul * qs) * g_ref[j]
        outs.append(intra + inter + og)
        st_ref[j] = st_ref[j] + jax.lax.dot_general(
            kj, vj, (((0,), (0,)), ((), ())), preferred_element_type=F32)

    y_ref[...] = x_ref[...] + jnp.concatenate(outs, axis=1)

    @pl.when(c == nc - 1)
    def _():
        fs_ref[...] = st_ref[...] + gf_ref[...]


def _mlp_up_body(x2_ref, w1_ref, w2_ref, a_ref):
    x2 = x2_ref[...]
    gate = jnp.dot(x2, w1_ref[...], preferred_element_type=F32)
    up = jnp.dot(x2, w2_ref[...], preferred_element_type=F32)
    a_ref[...] = (jax.nn.silu(up) * jax.nn.sigmoid(gate)).astype(BF16)


def _down_body(a_ref, wd_ref, y_ref, o_ref):
    o_ref[...] = y_ref[...] + jnp.dot(a_ref[...], wd_ref[...],
                                      preferred_element_type=F32)


def kernel(x, state, Wq, Wk, Wv, gamma, Wgate, Wdown, ln1_w, ln1_b,
           ln2_w, ln2_b):
    t, d = x.shape
    h, dh, _ = state.shape

    bt = min(256, t)
    bn = min(512, d)
    cs = min(256, t)
    nbm, nbn, nc, hp = t // bt, d // bn, t // cs, h // 2

    ln1w = ln1_w.reshape(1, d)
    ln1b = ln1_b.reshape(1, d)
    ln2w = ln2_w.reshape(1, d)
    ln2b = ln2_b.reshape(1, d)
    gamma3 = gamma.reshape(h, 1, dh)
    gfin = jnp.broadcast_to((t * gamma)[:, :, None], (h, dh, dh))
    wq16, wk16, wv16 = Wq.astype(BF16), Wk.astype(BF16), Wv.astype(BF16)
    wg16, wd16 = Wgate.astype(BF16), Wdown.astype(BF16)

    cp = pltpu.CompilerParams(
        dimension_semantics=("parallel", "arbitrary"),
        vmem_limit_bytes=100 * 1024 * 1024,
    )
    cp1 = pltpu.CompilerParams(
        dimension_semantics=("parallel",),
        vmem_limit_bytes=100 * 1024 * 1024,
    )

    row_spec = pl.BlockSpec((bt, d), lambda m: (m, 0))
    vec_spec = pl.BlockSpec((1, d), lambda m: (0, 0))

    xn = pl.pallas_call(
        _ln_body,
        grid=(nbm,),
        in_specs=[row_spec, vec_spec, vec_spec],
        out_specs=row_spec,
        out_shape=jax.ShapeDtypeStruct((t, d), BF16),
        compiler_params=cp1,
    )(x, ln1w, ln1b)

    q, k, v = pl.pallas_call(
        _qkv_body,
        grid=(nbn, nbm),
        in_specs=[
            pl.BlockSpec((bt, d), lambda n, m: (m, 0)),
            pl.BlockSpec((d, bn), lambda n, m: (0, n)),
            pl.BlockSpec((d, bn), lambda n, m: (0, n)),
            pl.BlockSpec((d, bn), lambda n, m: (0, n)),
        ],
        out_specs=[
            pl.BlockSpec((bt, bn), lambda n, m: (m, n)),
            pl.BlockSpec((bt, bn), lambda n, m: (m, n)),
            pl.BlockSpec((bt, bn), lambda n, m: (m, n)),
        ],
        out_shape=[jax.ShapeDtypeStruct((t, d), BF16)] * 3,
        compiler_params=cp,
    )(xn, wq16, wk16, wv16)

    y, fs = pl.pallas_call(
        functools.partial(_attn_body, nc, dh),
        grid=(hp, nc),
        in_specs=[
            pl.BlockSpec((cs, 2 * dh), lambda p, c: (c, p)),
            pl.BlockSpec((cs, 2 * dh), lambda p, c: (c, p)),
            pl.BlockSpec((cs, 2 * dh), lambda p, c: (c, p)),
            pl.BlockSpec((cs, 2 * dh), lambda p, c: (c, p)),
            pl.BlockSpec((2, 1, dh), lambda p, c: (p, 0, 0)),
            pl.BlockSpec((2, dh, dh), lambda p, c: (p, 0, 0)),
            pl.BlockSpec((2, dh, dh), lambda p, c: (p, 0, 0)),
        ],
        out_specs=[
            pl.BlockSpec((cs, 2 * dh), lambda p, c: (c, p)),
            pl.BlockSpec((2, dh, dh), lambda p, c: (p, 0, 0)),
        ],
        out_shape=[
            jax.ShapeDtypeStruct((t, d), F32),
            jax.ShapeDtypeStruct((h, dh, dh), F32),
        ],
        scratch_shapes=[pltpu.VMEM((2, dh, dh), F32)],
        compiler_params=cp,
    )(x, q, k, v, gamma3, state, gfin)

    x2 = pl.pallas_call(
        _ln_body,
        grid=(nbm,),
        in_specs=[row_spec, vec_spec, vec_spec],
        out_specs=row_spec,
        out_shape=jax.ShapeDtypeStruct((t, d), BF16),
        compiler_params=cp1,
    )(y, ln2w, ln2b)

    a = pl.pallas_call(
        _mlp_up_body,
        grid=(nbn, nbm),
        in_specs=[
            pl.BlockSpec((bt, d), lambda n, m: (m, 0)),
            pl.BlockSpec((d, bn), lambda n, m: (0, n)),
            pl.BlockSpec((d, bn), lambda n, m: (0, nbn + n)),
        ],
        out_specs=pl.BlockSpec((bt, bn), lambda n, m: (m, n)),
        out_shape=jax.ShapeDtypeStruct((t, d), BF16),
        compiler_params=cp,
    )(x2, wg16, wg16)

    out = pl.pallas_call(
        _down_body,
        grid=(nbn, nbm),
        in_specs=[
            pl.BlockSpec((bt, d), lambda n, m: (m, 0)),
            pl.BlockSpec((d, bn), lambda n, m: (0, n)),
            pl.BlockSpec((bt, bn), lambda n, m: (m, n)),
        ],
        out_specs=pl.BlockSpec((bt, bn), lambda n, m: (m, n)),
        out_shape=jax.ShapeDtypeStruct((t, d), F32),
        compiler_params=cp,
    )(a, wd16, y)

    return out, fs


# speedup vs baseline: 1.5768x; 1.5768x over previous
import functools

import jax
import jax.numpy as jnp
from jax.experimental import pallas as pl
from jax.experimental.pallas import tpu as pltpu

EPS = 1e-5
F32 = jnp.float32
BF16 = jnp.bfloat16


def _ln_rows(xr, w, b):
    m = jnp.mean(xr, axis=-1, keepdims=True)
    xc = xr - m
    v = jnp.mean(xc * xc, axis=-1, keepdims=True)
    return xc * jax.lax.rsqrt(v + EPS) * w + b


def _qkv_body(x_ref, w_ref, lw_ref, lb_ref, qkv_ref, xn_ref):
    xn_ref[...] = _ln_rows(x_ref[...], lw_ref[...], lb_ref[...]).astype(BF16)
    xn = xn_ref[...]
    n_out = w_ref.shape[1]
    for nb in range(0, n_out, 512):
        qkv_ref[:, nb:nb + 512] = jnp.dot(
            xn, w_ref[:, nb:nb + 512],
            preferred_element_type=F32).astype(BF16)


def _attn_body(nc, h, dh, x_ref, q_ref, k_ref, v_ref, e_ref, et_ref, g_ref,
               s0_ref, gf_ref, y_ref, fs_ref, st_ref):
    c = pl.program_id(0)

    @pl.when(c == 0)
    def _():
        st_ref[...] = s0_ref[...]

    cs = x_ref.shape[0]
    row = jax.lax.broadcasted_iota(jnp.int32, (cs, cs), 0)
    col = jax.lax.broadcasted_iota(jnp.int32, (cs, cs), 1)
    causal = row >= col
    tmul = (c * cs + 1 + jax.lax.broadcasted_iota(jnp.int32, (cs, 1), 0)
            ).astype(F32)
    qs_all = jnp.dot(q_ref[...], e_ref[...], preferred_element_type=F32)
    og_full = (tmul * jnp.dot(qs_all.astype(BF16), et_ref[...],
                              preferred_element_type=F32)) * g_ref[...]

    outs = []
    for j in range(h):
        qj = q_ref[:, j * dh:(j + 1) * dh]
        kj = k_ref[:, j * dh:(j + 1) * dh]
        vj = v_ref[:, j * dh:(j + 1) * dh]
        s = jax.lax.dot_general(qj, vj, (((1,), (1,)), ((), ())),
                                preferred_element_type=F32)
        sm = jnp.where(causal, s, 0.0).astype(BF16)
        intra = jnp.dot(sm, kj, preferred_element_type=F32)
        inter = jax.lax.dot_general(qj, st_ref[j].astype(BF16),
                                    (((1,), (1,)), ((), ())),
                                    preferred_element_type=F32)
        outs.append(intra + inter)
        st_ref[j] = st_ref[j] + jax.lax.dot_general(
            kj, vj, (((0,), (0,)), ((), ())), preferred_element_type=F32)

    y_ref[...] = x_ref[...] + og_full + jnp.concatenate(outs, axis=1)

    @pl.when(c == nc - 1)
    def _():
        fs_ref[...] = st_ref[...] + gf_ref[...]


def _mlp_up_body(d, y_ref, w_ref, lw_ref, lb_ref, a_ref, x2_ref):
    x2_ref[...] = _ln_rows(y_ref[...], lw_ref[...], lb_ref[...]).astype(BF16)
    x2 = x2_ref[...]
    for nb in range(0, d, 512):
        gate = jnp.dot(x2, w_ref[:, nb:nb + 512],
                       preferred_element_type=F32)
        up = jnp.dot(x2, w_ref[:, d + nb:d + nb + 512],
                     preferred_element_type=F32)
        a_ref[:, nb:nb + 512] = (
            up / ((1.0 + jnp.exp(-up)) * (1.0 + jnp.exp(-gate)))
        ).astype(BF16)


def _down_body(a_ref, wd_ref, y_ref, o_ref):
    a = a_ref[...]
    n_out = wd_ref.shape[1]
    for nb in range(0, n_out, 512):
        o_ref[:, nb:nb + 512] = (
            y_ref[:, nb:nb + 512]
            + jnp.dot(a, wd_ref[:, nb:nb + 512],
                      preferred_element_type=F32))


def kernel(x, state, Wq, Wk, Wv, gamma, Wgate, Wdown, ln1_w, ln1_b,
           ln2_w, ln2_b):
    t, d = x.shape
    h, dh, _ = state.shape

    bt = min(512, t)
    cs = min(256, t)
    nbm, nc = t // bt, t // cs

    ln1w = ln1_w.reshape(1, d)
    ln1b = ln1_b.reshape(1, d)
    ln2w = ln2_w.reshape(1, d)
    ln2b = ln2_b.reshape(1, d)
    grow = gamma.reshape(1, d)
    lane = jnp.arange(d, dtype=jnp.int32)
    emat = (lane[:, None] // dh == jnp.arange(h, dtype=jnp.int32)[None, :]
            ).astype(BF16)
    etmat = (jnp.arange(h, dtype=jnp.int32)[:, None] == lane[None, :] // dh
             ).astype(BF16)
    gfin = jnp.broadcast_to((t * gamma)[:, :, None], (h, dh, dh))
    wqkv16 = jnp.concatenate(
        [Wq.astype(BF16), Wk.astype(BF16), Wv.astype(BF16)], axis=1)
    wg16, wd16 = Wgate.astype(BF16), Wdown.astype(BF16)

    cp = pltpu.CompilerParams(
        dimension_semantics=("arbitrary",),
        vmem_limit_bytes=100 * 1024 * 1024,
    )

    qkv = pl.pallas_call(
        _qkv_body,
        grid=(nbm,),
        in_specs=[
            pl.BlockSpec((bt, d), lambda m: (m, 0)),
            pl.BlockSpec((d, 3 * d), lambda m: (0, 0)),
            pl.BlockSpec((1, d), lambda m: (0, 0)),
            pl.BlockSpec((1, d), lambda m: (0, 0)),
        ],
        out_specs=pl.BlockSpec((bt, 3 * d), lambda m: (m, 0)),
        out_shape=jax.ShapeDtypeStruct((t, 3 * d), BF16),
        scratch_shapes=[pltpu.VMEM((bt, d), BF16)],
        compiler_params=cp,
    )(x, wqkv16, ln1w, ln1b)

    y, fs = pl.pallas_call(
        functools.partial(_attn_body, nc, h, dh),
        grid=(nc,),
        in_specs=[
            pl.BlockSpec((cs, d), lambda c: (c, 0)),
            pl.BlockSpec((cs, d), lambda c: (c, 0)),
            pl.BlockSpec((cs, d), lambda c: (c, 1)),
            pl.BlockSpec((cs, d), lambda c: (c, 2)),
            pl.BlockSpec((d, h), lambda c: (0, 0)),
            pl.BlockSpec((h, d), lambda c: (0, 0)),
            pl.BlockSpec((1, d), lambda c: (0, 0)),
            pl.BlockSpec((h, dh, dh), lambda c: (0, 0, 0)),
            pl.BlockSpec((h, dh, dh), lambda c: (0, 0, 0)),
        ],
        out_specs=[
            pl.BlockSpec((cs, d), lambda c: (c, 0)),
            pl.BlockSpec((h, dh, dh), lambda c: (0, 0, 0)),
        ],
        out_shape=[
            jax.ShapeDtypeStruct((t, d), F32),
            jax.ShapeDtypeStruct((h, dh, dh), F32),
        ],
        scratch_shapes=[pltpu.VMEM((h, dh, dh), F32)],
        compiler_params=cp,
    )(x, qkv, qkv, qkv, emat, etmat, grow, state, gfin)

    a = pl.pallas_call(
        functools.partial(_mlp_up_body, d),
        grid=(nbm,),
        in_specs=[
            pl.BlockSpec((bt, d), lambda m: (m, 0)),
            pl.BlockSpec((d, 2 * d), lambda m: (0, 0)),
            pl.BlockSpec((1, d), lambda m: (0, 0)),
            pl.BlockSpec((1, d), lambda m: (0, 0)),
        ],
        out_specs=pl.BlockSpec((bt, d), lambda m: (m, 0)),
        out_shape=jax.ShapeDtypeStruct((t, d), BF16),
        scratch_shapes=[pltpu.VMEM((bt, d), BF16)],
        compiler_params=cp,
    )(y, wg16, ln2w, ln2b)

    out = pl.pallas_call(
        _down_body,
        grid=(nbm,),
        in_specs=[
            pl.BlockSpec((bt, d), lambda m: (m, 0)),
            pl.BlockSpec((d, d), lambda m: (0, 0)),
            pl.BlockSpec((bt, d), lambda m: (m, 0)),
        ],
        out_specs=pl.BlockSpec((bt, d), lambda m: (m, 0)),
        out_shape=jax.ShapeDtypeStruct((t, d), F32),
        compiler_params=cp,
    )(a, wd16, y)

    return out, fs


# speedup vs baseline: 1.6092x; 1.0205x over previous
import functools

import jax
import jax.numpy as jnp
from jax.experimental import pallas as pl
from jax.experimental.pallas import tpu as pltpu

EPS = 1e-5
F32 = jnp.float32
BF16 = jnp.bfloat16


def _ln_rows(xr, w, b):
    m = jnp.mean(xr, axis=-1, keepdims=True)
    xc = xr - m
    v = jnp.mean(xc * xc, axis=-1, keepdims=True)
    return xc * jax.lax.rsqrt(v + EPS) * w + b


def _qkv_body(x_ref, w_ref, lw_ref, lb_ref, qkv_ref, xn_ref):
    xn_ref[...] = _ln_rows(x_ref[...], lw_ref[...], lb_ref[...]).astype(BF16)
    xn = xn_ref[...]
    n_out = w_ref.shape[1]
    for nb in range(0, n_out, 512):
        qkv_ref[:, nb:nb + 512] = jnp.dot(
            xn, w_ref[:, nb:nb + 512],
            preferred_element_type=F32).astype(BF16)


def _attn_body(nc, h, dh, x_ref, q_ref, k_ref, v_ref, e_ref, et_ref, g_ref,
               s0_ref, gf_ref, y_ref, fs_ref, st_ref):
    c = pl.program_id(0)

    @pl.when(c == 0)
    def _():
        st_ref[...] = s0_ref[...]

    cs = x_ref.shape[0]
    row = jax.lax.broadcasted_iota(jnp.int32, (cs, cs), 0)
    col = jax.lax.broadcasted_iota(jnp.int32, (cs, cs), 1)
    causal = row >= col
    tmul = (c * cs + 1 + jax.lax.broadcasted_iota(jnp.int32, (cs, 1), 0)
            ).astype(F32)
    qs_all = jnp.dot(q_ref[...], e_ref[...], preferred_element_type=F32)
    og_full = (tmul * jnp.dot(qs_all.astype(BF16), et_ref[...],
                              preferred_element_type=F32)) * g_ref[...]

    outs = []
    for j in range(h):
        qj = q_ref[:, j * dh:(j + 1) * dh]
        kj = k_ref[:, j * dh:(j + 1) * dh]
        vj = v_ref[:, j * dh:(j + 1) * dh]
        s = jax.lax.dot_general(qj, vj, (((1,), (1,)), ((), ())),
                                preferred_element_type=F32)
        sm = jnp.where(causal, s, 0.0).astype(BF16)
        intra = jnp.dot(sm, kj, preferred_element_type=F32)
        inter = jax.lax.dot_general(qj, st_ref[j].astype(BF16),
                                    (((1,), (1,)), ((), ())),
                                    preferred_element_type=F32)
        outs.append(intra + inter)
        st_ref[j] = st_ref[j] + jax.lax.dot_general(
            kj, vj, (((0,), (0,)), ((), ())), preferred_element_type=F32)

    y_ref[...] = x_ref[...] + og_full + jnp.concatenate(outs, axis=1)

    @pl.when(c == nc - 1)
    def _():
        fs_ref[...] = st_ref[...] + gf_ref[...]


def _mlp_body(d, y_ref, w_ref, wd_ref, lw_ref, lb_ref, o_ref, x2_ref,
              a_ref):
    x2_ref[...] = _ln_rows(y_ref[...], lw_ref[...], lb_ref[...]).astype(BF16)
    x2 = x2_ref[...]
    for nb in range(0, d, 512):
        gate = jnp.dot(x2, w_ref[:, nb:nb + 512],
                       preferred_element_type=F32)
        up = jnp.dot(x2, w_ref[:, d + nb:d + nb + 512],
                     preferred_element_type=F32)
        a_ref[:, nb:nb + 512] = (
            up / ((1.0 + jnp.exp(-up)) * (1.0 + jnp.exp(-gate)))
        ).astype(BF16)
    a = a_ref[...]
    for nb in range(0, d, 512):
        o_ref[:, nb:nb + 512] = (
            y_ref[:, nb:nb + 512]
            + jnp.dot(a, wd_ref[:, nb:nb + 512],
                      preferred_element_type=F32))


def kernel(x, state, Wq, Wk, Wv, gamma, Wgate, Wdown, ln1_w, ln1_b,
           ln2_w, ln2_b):
    t, d = x.shape
    h, dh, _ = state.shape

    bt = min(512, t)
    cs = min(256, t)
    nbm, nc = t // bt, t // cs

    ln1w = ln1_w.reshape(1, d)
    ln1b = ln1_b.reshape(1, d)
    ln2w = ln2_w.reshape(1, d)
    ln2b = ln2_b.reshape(1, d)
    grow = gamma.reshape(1, d)
    lane = jnp.arange(d, dtype=jnp.int32)
    emat = (lane[:, None] // dh == jnp.arange(h, dtype=jnp.int32)[None, :]
            ).astype(BF16)
    etmat = (jnp.arange(h, dtype=jnp.int32)[:, None] == lane[None, :] // dh
             ).astype(BF16)
    gfin = jnp.broadcast_to((t * gamma)[:, :, None], (h, dh, dh))
    wqkv16 = jnp.concatenate(
        [Wq.astype(BF16), Wk.astype(BF16), Wv.astype(BF16)], axis=1)
    wg16, wd16 = Wgate.astype(BF16), Wdown.astype(BF16)

    cp = pltpu.CompilerParams(
        dimension_semantics=("arbitrary",),
        vmem_limit_bytes=100 * 1024 * 1024,
    )

    qkv = pl.pallas_call(
        _qkv_body,
        grid=(nbm,),
        in_specs=[
            pl.BlockSpec((bt, d), lambda m: (m, 0)),
            pl.BlockSpec((d, 3 * d), lambda m: (0, 0)),
            pl.BlockSpec((1, d), lambda m: (0, 0)),
            pl.BlockSpec((1, d), lambda m: (0, 0)),
        ],
        out_specs=pl.BlockSpec((bt, 3 * d), lambda m: (m, 0)),
        out_shape=jax.ShapeDtypeStruct((t, 3 * d), BF16),
        scratch_shapes=[pltpu.VMEM((bt, d), BF16)],
        compiler_params=cp,
    )(x, wqkv16, ln1w, ln1b)

    y, fs = pl.pallas_call(
        functools.partial(_attn_body, nc, h, dh),
        grid=(nc,),
        in_specs=[
            pl.BlockSpec((cs, d), lambda c: (c, 0)),
            pl.BlockSpec((cs, d), lambda c: (c, 0)),
            pl.BlockSpec((cs, d), lambda c: (c, 1)),
            pl.BlockSpec((cs, d), lambda c: (c, 2)),
            pl.BlockSpec((d, h), lambda c: (0, 0)),
            pl.BlockSpec((h, d), lambda c: (0, 0)),
            pl.BlockSpec((1, d), lambda c: (0, 0)),
            pl.BlockSpec((h, dh, dh), lambda c: (0, 0, 0)),
            pl.BlockSpec((h, dh, dh), lambda c: (0, 0, 0)),
        ],
        out_specs=[
            pl.BlockSpec((cs, d), lambda c: (c, 0)),
            pl.BlockSpec((h, dh, dh), lambda c: (0, 0, 0)),
        ],
        out_shape=[
            jax.ShapeDtypeStruct((t, d), F32),
            jax.ShapeDtypeStruct((h, dh, dh), F32),
        ],
        scratch_shapes=[pltpu.VMEM((h, dh, dh), F32)],
        compiler_params=cp,
    )(x, qkv, qkv, qkv, emat, etmat, grow, state, gfin)

    out = pl.pallas_call(
        functools.partial(_mlp_body, d),
        grid=(nbm,),
        in_specs=[
            pl.BlockSpec((bt, d), lambda m: (m, 0)),
            pl.BlockSpec((d, 2 * d), lambda m: (0, 0)),
            pl.BlockSpec((d, d), lambda m: (0, 0)),
            pl.BlockSpec((1, d), lambda m: (0, 0)),
            pl.BlockSpec((1, d), lambda m: (0, 0)),
        ],
        out_specs=pl.BlockSpec((bt, d), lambda m: (m, 0)),
        out_shape=jax.ShapeDtypeStruct((t, d), F32),
        scratch_shapes=[pltpu.VMEM((bt, d), BF16),
                        pltpu.VMEM((bt, d), BF16)],
        compiler_params=cp,
    )(y, wg16, wd16, ln2w, ln2b)

    return out, fs


# speedup vs baseline: 1.6543x; 1.0281x over previous
import functools

import jax
import jax.numpy as jnp
from jax.experimental import pallas as pl
from jax.experimental.pallas import tpu as pltpu

EPS = 1e-5
F32 = jnp.float32
BF16 = jnp.bfloat16


def _ln_rows(xr, w, b):
    m = jnp.mean(xr, axis=-1, keepdims=True)
    xc = xr - m
    v = jnp.mean(xc * xc, axis=-1, keepdims=True)
    return xc * jax.lax.rsqrt(v + EPS) * w + b


def _attn_fused_body(h, dh, ncs, x_ref, w_ref, lw_ref, lb_ref, e_ref,
                     et_ref, g_ref, s0_ref, gf_ref, y_ref, fs_ref,
                     xn_ref, qkv_ref, st_ref):
    m = pl.program_id(0)
    nbm = pl.num_programs(0)

    @pl.when(m == 0)
    def _():
        st_ref[...] = s0_ref[...]

    bt = x_ref.shape[0]
    d = lw_ref.shape[1]
    xn_ref[...] = _ln_rows(x_ref[...], lw_ref[...], lb_ref[...]).astype(BF16)
    xn = xn_ref[...]
    for nb in range(0, 3 * d, 512):
        qkv_ref[:, nb:nb + 512] = jnp.dot(
            xn, w_ref[:, nb:nb + 512],
            preferred_element_type=F32).astype(BF16)

    cs = bt // ncs
    row = jax.lax.broadcasted_iota(jnp.int32, (cs, cs), 0)
    col = jax.lax.broadcasted_iota(jnp.int32, (cs, cs), 1)
    causal = row >= col
    riota = jax.lax.broadcasted_iota(jnp.int32, (cs, 1), 0)

    for sub in range(ncs):
        base = sub * cs
        tmul = ((m * ncs + sub) * cs + 1 + riota).astype(F32)
        qf = qkv_ref[base:base + cs, 0:d]
        kf = qkv_ref[base:base + cs, d:2 * d]
        vf = qkv_ref[base:base + cs, 2 * d:3 * d]
        qs_all = jnp.dot(qf, e_ref[...], preferred_element_type=F32)
        og_full = (tmul * jnp.dot(qs_all.astype(BF16), et_ref[...],
                                  preferred_element_type=F32)) * g_ref[...]
        outs = []
        for j in range(h):
            qj = qf[:, j * dh:(j + 1) * dh]
            kj = kf[:, j * dh:(j + 1) * dh]
            vj = vf[:, j * dh:(j + 1) * dh]
            s = jax.lax.dot_general(qj, vj, (((1,), (1,)), ((), ())),
                                    preferred_element_type=F32)
            sm = jnp.where(causal, s, 0.0).astype(BF16)
            intra = jnp.dot(sm, kj, preferred_element_type=F32)
            inter = jax.lax.dot_general(qj, st_ref[j].astype(BF16),
                                        (((1,), (1,)), ((), ())),
                                        preferred_element_type=F32)
            outs.append(intra + inter)
            st_ref[j] = st_ref[j] + jax.lax.dot_general(
                kj, vj, (((0,), (0,)), ((), ())),
                preferred_element_type=F32)
        y_ref[base:base + cs, :] = (x_ref[base:base + cs, :] + og_full
                                    + jnp.concatenate(outs, axis=1))

    @pl.when(m == nbm - 1)
    def _():
        fs_ref[...] = st_ref[...] + gf_ref[...]


def _mlp_body(d, y_ref, w_ref, wd_ref, lw_ref, lb_ref, o_ref, x2_ref,
              a_ref):
    x2_ref[...] = _ln_rows(y_ref[...], lw_ref[...], lb_ref[...]).astype(BF16)
    x2 = x2_ref[...]
    for nb in range(0, d, 512):
        gate = jnp.dot(x2, w_ref[:, nb:nb + 512],
                       preferred_element_type=F32)
        up = jnp.dot(x2, w_ref[:, d + nb:d + nb + 512],
                     preferred_element_type=F32)
        a_ref[:, nb:nb + 512] = (
            up / ((1.0 + jnp.exp(-up)) * (1.0 + jnp.exp(-gate)))
        ).astype(BF16)
    a = a_ref[...]
    for nb in range(0, d, 512):
        o_ref[:, nb:nb + 512] = (
            y_ref[:, nb:nb + 512]
            + jnp.dot(a, wd_ref[:, nb:nb + 512],
                      preferred_element_type=F32))


def kernel(x, state, Wq, Wk, Wv, gamma, Wgate, Wdown, ln1_w, ln1_b,
           ln2_w, ln2_b):
    t, d = x.shape
    h, dh, _ = state.shape

    bt = min(512, t)
    cs = min(256, t)
    nbm, nc = t // bt, t // cs

    ln1w = ln1_w.reshape(1, d)
    ln1b = ln1_b.reshape(1, d)
    ln2w = ln2_w.reshape(1, d)
    ln2b = ln2_b.reshape(1, d)
    grow = gamma.reshape(1, d)
    lane = jnp.arange(d, dtype=jnp.int32)
    emat = (lane[:, None] // dh == jnp.arange(h, dtype=jnp.int32)[None, :]
            ).astype(BF16)
    etmat = (jnp.arange(h, dtype=jnp.int32)[:, None] == lane[None, :] // dh
             ).astype(BF16)
    gfin = jnp.broadcast_to((t * gamma)[:, :, None], (h, dh, dh))
    wqkv16 = jnp.concatenate(
        [Wq.astype(BF16), Wk.astype(BF16), Wv.astype(BF16)], axis=1)
    wg16, wd16 = Wgate.astype(BF16), Wdown.astype(BF16)

    cp = pltpu.CompilerParams(
        dimension_semantics=("arbitrary",),
        vmem_limit_bytes=100 * 1024 * 1024,
    )

    ncs = bt // cs
    y, fs = pl.pallas_call(
        functools.partial(_attn_fused_body, h, dh, ncs),
        grid=(nbm,),
        in_specs=[
            pl.BlockSpec((bt, d), lambda m: (m, 0)),
            pl.BlockSpec((d, 3 * d), lambda m: (0, 0)),
            pl.BlockSpec((1, d), lambda m: (0, 0)),
            pl.BlockSpec((1, d), lambda m: (0, 0)),
            pl.BlockSpec((d, h), lambda m: (0, 0)),
            pl.BlockSpec((h, d), lambda m: (0, 0)),
            pl.BlockSpec((1, d), lambda m: (0, 0)),
            pl.BlockSpec((h, dh, dh), lambda m: (0, 0, 0)),
            pl.BlockSpec((h, dh, dh), lambda m: (0, 0, 0)),
        ],
        out_specs=[
            pl.BlockSpec((bt, d), lambda m: (m, 0)),
            pl.BlockSpec((h, dh, dh), lambda m: (0, 0, 0)),
        ],
        out_shape=[
            jax.ShapeDtypeStruct((t, d), F32),
            jax.ShapeDtypeStruct((h, dh, dh), F32),
        ],
        scratch_shapes=[
            pltpu.VMEM((bt, d), BF16),
            pltpu.VMEM((bt, 3 * d), BF16),
            pltpu.VMEM((h, dh, dh), F32),
        ],
        compiler_params=cp,
    )(x, wqkv16, ln1w, ln1b, emat, etmat, grow, state, gfin)

    out = pl.pallas_call(
        functools.partial(_mlp_body, d),
        grid=(nbm,),
        in_specs=[
            pl.BlockSpec((bt, d), lambda m: (m, 0)),
            pl.BlockSpec((d, 2 * d), lambda m: (0, 0)),
            pl.BlockSpec((d, d), lambda m: (0, 0)),
            pl.BlockSpec((1, d), lambda m: (0, 0)),
            pl.BlockSpec((1, d), lambda m: (0, 0)),
        ],
        out_specs=pl.BlockSpec((bt, d), lambda m: (m, 0)),
        out_shape=jax.ShapeDtypeStruct((t, d), F32),
        scratch_shapes=[pltpu.VMEM((bt, d), BF16),
                        pltpu.VMEM((bt, d), BF16)],
        compiler_params=cp,
    )(y, wg16, wd16, ln2w, ln2b)

    return out, fs
